# initial kernel scaffold (unmeasured)
import jax
import jax.numpy as jnp
from jax import lax
from jax.experimental import pallas as pl
from jax.experimental.pallas import tpu as pltpu

B, H, D, BS = 32, 16, 128, 32
NSLOTS = 256
NPAGES_LOCAL = 256
CHUNK = 32
NC = NPAGES_LOCAL // CHUNK
NK = CHUNK * BS
SCALE = D ** -0.5


def kernel(Q, K, V, bt, lens):
    lens2 = lens.reshape(B, 1)

    def body(q_ref, k_ref, v_ref, bt_ref, lens_ref, out_ref,
             acc_o, acc_l, recv_o, recv_l, send_sems, recv_sems):
        c = pl.program_id(0)
        my_x = lax.axis_index("x")
        my_y = lax.axis_index("y")

        @pl.when(c == 0)
        def _init():
            acc_o[...] = jnp.zeros_like(acc_o)
            acc_l[...] = jnp.zeros_like(acc_l)
            barrier = pltpu.get_barrier_semaphore()
            pl.semaphore_signal(
                barrier, inc=1,
                device_id=(my_x, 1 - my_y),
                device_id_type=pl.DeviceIdType.MESH,
            )
            pl.semaphore_wait(barrier, 1)

        page_ids = (
            my_y * NPAGES_LOCAL + c * CHUNK
            + lax.broadcasted_iota(jnp.int32, (1, CHUNK, 1), 1)
        )
        slot = lax.broadcasted_iota(jnp.int32, (1, 1, NSLOTS), 2)
        hit = (bt_ref[...][:, None, :] == page_ids) & (
            slot < lens_ref[...][:, :, None]
        )
        w_page = jnp.sum(jnp.where(hit, 1.0, 0.0), axis=2)
        w_keys = jnp.broadcast_to(
            w_page[:, :, None], (B, CHUNK, BS)
        ).reshape(B, NK)

        for h in range(H):
            qh = q_ref[:, 0, h, :]
            kh = k_ref[:, :, h, :].reshape(NK, D)
            vh = v_ref[:, :, h, :].reshape(NK, D)
            s = lax.dot_general(
                qh, kh, (((1,), (1,)), ((), ())),
                preferred_element_type=jnp.float32,
            ) * SCALE
            p = jnp.exp(s) * w_keys
            acc_l[h, :] += jnp.sum(p, axis=1)
            acc_o[h, :, :] += lax.dot_general(
                p, vh, (((1,), (0,)), ((), ())),
                preferred_element_type=jnp.float32,
            )

        @pl.when(c == NC - 1)
        def _exchange():
            nbr = (my_x, 1 - my_y)
            rdma_o = pltpu.make_async_remote_copy(
                src_ref=acc_o, dst_ref=recv_o,
                send_sem=send_sems.at[0], recv_sem=recv_sems.at[0],
                device_id=nbr, device_id_type=pl.DeviceIdType.MESH,
            )
            rdma_l = pltpu.make_async_remote_copy(
                src_ref=acc_l, dst_ref=recv_l,
                send_sem=send_sems.at[1], recv_sem=recv_sems.at[1],
                device_id=nbr, device_id_type=pl.DeviceIdType.MESH,
            )
            rdma_o.start()
            rdma_l.start()
            rdma_o.wait()
            rdma_l.wait()
            l_tot = acc_l[...] + recv_l[...]
            o_tot = acc_o[...] + recv_o[...]
            o = o_tot / l_tot[:, :, None]
            out_ref[...] = jnp.transpose(o, (1, 0, 2))[:, None, :, :]

    return pl.pallas_call(
        body,
        grid=(NC,),
        in_specs=[
            pl.BlockSpec((B, 1, H, D), lambda c: (0, 0, 0, 0)),
            pl.BlockSpec((CHUNK, BS, H, D), lambda c: (c, 0, 0, 0)),
            pl.BlockSpec((CHUNK, BS, H, D), lambda c: (c, 0, 0, 0)),
            pl.BlockSpec((B, NSLOTS), lambda c: (0, 0)),
            pl.BlockSpec((B, 1), lambda c: (0, 0)),
        ],
        out_specs=pl.BlockSpec((B, 1, H, D), lambda c: (0, 0, 0, 0)),
        out_shape=jax.ShapeDtypeStruct((B, 1, H, D), jnp.float32),
        scratch_shapes=[
            pltpu.VMEM((H, B, D), jnp.float32),
            pltpu.VMEM((H, B), jnp.float32),
            pltpu.VMEM((H, B, D), jnp.float32),
            pltpu.VMEM((H, B), jnp.float32),
            pltpu.SemaphoreType.DMA((2,)),
            pltpu.SemaphoreType.DMA((2,)),
        ],
        compiler_params=pltpu.CompilerParams(
            collective_id=0,
            dimension_semantics=("arbitrary",),
        ),
    )(Q, K, V, bt, lens2)


# baseline (device time: 140244 ns/iter reference)
import jax
import jax.numpy as jnp
from jax import lax
from jax.experimental import pallas as pl
from jax.experimental.pallas import tpu as pltpu

B, H, D, BS = 32, 16, 128, 32
NSLOTS = 256
NPAGES_LOCAL = 256
CHUNK = 16
NC = NPAGES_LOCAL // CHUNK
NK = CHUNK * BS
SCALE = D ** -0.5


def kernel(Q, K, V, bt, lens):
    lens2 = lens.reshape(B, 1)

    def body(q_ref, k_ref, v_ref, bt_ref, lens_ref, out_ref,
             acc_o, acc_l, recv_o, recv_l, send_sems, recv_sems):
        c = pl.program_id(0)
        my_x = lax.axis_index("x")
        my_y = lax.axis_index("y")

        @pl.when(c == 0)
        def _init():
            acc_o[...] = jnp.zeros_like(acc_o)
            acc_l[...] = jnp.zeros_like(acc_l)
            barrier = pltpu.get_barrier_semaphore()
            pl.semaphore_signal(
                barrier, inc=1,
                device_id=(my_x, 1 - my_y),
                device_id_type=pl.DeviceIdType.MESH,
            )
            pl.semaphore_wait(barrier, 1)

        page_ids = (
            my_y * NPAGES_LOCAL + c * CHUNK
            + lax.broadcasted_iota(jnp.int32, (1, CHUNK, 1), 1)
        )
        slot = lax.broadcasted_iota(jnp.int32, (1, 1, NSLOTS), 2)
        hit = (bt_ref[...][:, None, :] == page_ids) & (
            slot < lens_ref[...][:, :, None]
        )
        w_page = jnp.sum(jnp.where(hit, 1.0, 0.0), axis=2)
        w_keys = jnp.broadcast_to(
            w_page[:, :, None], (B, CHUNK, BS)
        ).reshape(B, NK)

        for h in range(H):
            qh = q_ref[:, 0, h, :]
            kh = k_ref[:, :, h, :].reshape(NK, D)
            vh = v_ref[:, :, h, :].reshape(NK, D)
            s = lax.dot_general(
                qh, kh, (((1,), (1,)), ((), ())),
                preferred_element_type=jnp.float32,
            ) * SCALE
            p = jnp.exp(s) * w_keys
            acc_l[h, :] += jnp.sum(p, axis=1)
            acc_o[h, :, :] += lax.dot_general(
                p, vh, (((1,), (0,)), ((), ())),
                preferred_element_type=jnp.float32,
            )

        @pl.when(c == NC - 1)
        def _exchange():
            nbr = (my_x, 1 - my_y)
            rdma_o = pltpu.make_async_remote_copy(
                src_ref=acc_o, dst_ref=recv_o,
                send_sem=send_sems.at[0], recv_sem=recv_sems.at[0],
                device_id=nbr, device_id_type=pl.DeviceIdType.MESH,
            )
            rdma_l = pltpu.make_async_remote_copy(
                src_ref=acc_l, dst_ref=recv_l,
                send_sem=send_sems.at[1], recv_sem=recv_sems.at[1],
                device_id=nbr, device_id_type=pl.DeviceIdType.MESH,
            )
            rdma_o.start()
            rdma_l.start()
            rdma_o.wait()
            rdma_l.wait()
            l_tot = acc_l[...] + recv_l[...]
            o_tot = acc_o[...] + recv_o[...]
            o = o_tot / l_tot[:, :, None]
            out_ref[...] = jnp.transpose(o, (1, 0, 2))[:, None, :, :]

    return pl.pallas_call(
        body,
        grid=(NC,),
        in_specs=[
            pl.BlockSpec((B, 1, H, D), lambda c: (0, 0, 0, 0)),
            pl.BlockSpec((CHUNK, BS, H, D), lambda c: (c, 0, 0, 0)),
            pl.BlockSpec((CHUNK, BS, H, D), lambda c: (c, 0, 0, 0)),
            pl.BlockSpec((B, NSLOTS), lambda c: (0, 0)),
            pl.BlockSpec((B, 1), lambda c: (0, 0)),
        ],
        out_specs=pl.BlockSpec((B, 1, H, D), lambda c: (0, 0, 0, 0)),
        out_shape=jax.ShapeDtypeStruct((B, 1, H, D), jnp.float32),
        scratch_shapes=[
            pltpu.VMEM((H, B, D), jnp.float32),
            pltpu.VMEM((H, B), jnp.float32),
            pltpu.VMEM((H, B, D), jnp.float32),
            pltpu.VMEM((H, B), jnp.float32),
            pltpu.SemaphoreType.DMA((2,)),
            pltpu.SemaphoreType.DMA((2,)),
        ],
        compiler_params=pltpu.CompilerParams(
            collective_id=0,
            dimension_semantics=("arbitrary",),
        ),
    )(Q, K, V, bt, lens2)
